# baseline (speedup 1.0000x reference)
import functools

import jax
import jax.numpy as jnp
from jax import lax
from jax.experimental import pallas as pl
from jax.experimental.pallas import tpu as pltpu
from jax.experimental.pallas import tpu_sc as plsc

N = 10000
E = 320000
D = 128
H = 128
C = 40

NC = 2
NS = 16
NTILES = NC * NS

NPAD = 10240
EPT = 10240
EPAD = NTILES * EPT
CHUNK = 128
EPT2 = EPAD // NS
NCH2 = EPT2 // CHUNK
ROWS_PER_TILE = NPAD // NS

HH = H // 2
CP = 64
CH = CP // 2
RB = 1024
GRID = NPAD // RB

_mesh = plsc.VectorSubcoreMesh(core_axis_name="c", subcore_axis_name="s")


@functools.partial(
    pl.kernel,
    out_type=jax.ShapeDtypeStruct((NTILES, NPAD), jnp.float32),
    mesh=_mesh,
    scratch_types=[
        pltpu.VMEM((EPT,), jnp.int32),
        pltpu.VMEM((NPAD,), jnp.float32),
    ],
    compiler_params=pltpu.CompilerParams(needs_layout_passes=False),
)
def _deg_kernel(dst_hbm, out_hbm, dst_v, hist_v):
    g = lax.axis_index("c") * NS + lax.axis_index("s")
    pltpu.sync_copy(dst_hbm.at[g], dst_v)
    zero16 = jnp.zeros((16,), jnp.float32)
    ones16 = jnp.ones((16,), jnp.float32)

    def zbody(i, _):
        hist_v[pl.ds(i * 16, 16)] = zero16
        return 0

    lax.fori_loop(0, NPAD // 16, zbody, 0)

    def hbody(i, _):
        idx = dst_v[pl.ds(i * 16, 16)]
        plsc.addupdate_scatter(hist_v, [idx], ones16)
        return 0

    lax.fori_loop(0, EPT // 16, hbody, 0)
    pltpu.sync_copy(hist_v, out_hbm.at[g])


def _make_scatter(width, G, chunk):
    nch = EPT2 // chunk
    ngroups = nch // G
    npairs = ngroups // 2
    assert EPT2 % chunk == 0 and nch % G == 0 and ngroups % 2 == 0
    assert npairs >= 2

    @functools.partial(
        pl.kernel,
        out_type=jax.ShapeDtypeStruct((NC, NPAD, width), jnp.float32),
        mesh=_mesh,
        scratch_types=[
            pltpu.VMEM((nch, chunk), jnp.int32),
            pltpu.VMEM((nch, chunk), jnp.int32),
            pltpu.VMEM((2 * G, chunk, width), jnp.float32),
            pltpu.VMEM_SHARED((NPAD, width), jnp.float32),
            pltpu.SemaphoreType.DMA,
            pltpu.SemaphoreType.DMA,
            pltpu.SemaphoreType.DMA,
            pltpu.SemaphoreType.DMA,
        ],
        compiler_params=pltpu.CompilerParams(use_tc_tiling_on_sc=False),
    )
    def _scatter(p_hbm, src_hbm, dst_hbm, out_hbm,
                 src_v, dst_v, rows_v, agg_sh, gs0, gs1, ss0, ss1):
        c = lax.axis_index("c")
        s = lax.axis_index("s")
        gsems = (gs0, gs1)
        ssems = (ss0, ss1)
        pltpu.sync_copy(src_hbm.at[c, s], src_v)
        pltpu.sync_copy(dst_hbm.at[s], dst_v)
        base = s * ROWS_PER_TILE
        pltpu.sync_copy(p_hbm.at[pl.ds(c * NPAD + base, ROWS_PER_TILE)],
                        agg_sh.at[pl.ds(base, ROWS_PER_TILE)])
        plsc.subcore_barrier()

        def fire_gathers(t, st):
            for b in range(G):
                pltpu.async_copy(p_hbm.at[src_v.at[t * G + b]],
                                 rows_v.at[st * G + b], gsems[st])

        def wait_gathers(t, st):
            for b in range(G):
                pltpu.make_async_copy(p_hbm.at[src_v.at[t * G + b]],
                                      rows_v.at[st * G + b], gsems[st]).wait()

        def fire_scatters(t, st):
            for b in range(G):
                pltpu.async_copy(rows_v.at[st * G + b],
                                 agg_sh.at[dst_v.at[t * G + b]], ssems[st],
                                 add=True)

        def wait_scatters(t, st):
            for b in range(G):
                pltpu.make_async_copy(rows_v.at[st * G + b],
                                      agg_sh.at[dst_v.at[t * G + b]],
                                      ssems[st]).wait()

        fire_gathers(0, 0)

        def obody(u, _):
            t0 = 2 * u
            t1 = 2 * u + 1
            wait_gathers(t0, 0)
            fire_scatters(t0, 0)

            @pl.when(u > 0)
            def _():
                wait_scatters(t1 - 2, 1)

            fire_gathers(t1, 1)
            wait_gathers(t1, 1)
            fire_scatters(t1, 1)
            wait_scatters(t0, 0)

            @pl.when(u < npairs - 1)
            def _():
                fire_gathers(t0 + 2, 0)

            return 0

        lax.fori_loop(0, npairs, obody, 0)
        wait_scatters(2 * npairs - 1, 1)
        plsc.subcore_barrier()
        pltpu.sync_copy(agg_sh.at[pl.ds(base, ROWS_PER_TILE)],
                        out_hbm.at[c, pl.ds(base, ROWS_PER_TILE)])

    return _scatter


CH1 = 80
CH2 = 64
_scatter_l1 = _make_scatter(HH, 4, CH1)
_scatter_l2 = _make_scatter(CH, 16, CH2)


def _dinv_of(deg_ref):
    deg = jnp.sum(deg_ref[...], axis=0) + 1.0
    return lax.rsqrt(deg)


def _tc1_body(x_ref, w_ref, deg_ref, o_ref):
    dinv = _dinv_of(deg_ref)
    h = jnp.dot(x_ref[...], w_ref[0], preferred_element_type=jnp.float32)
    o_ref[...] = h * dinv[:, None]


def _tc1(x_p, w1h, deg_parts):
    return pl.pallas_call(
        _tc1_body,
        grid=(2 * GRID,),
        in_specs=[
            pl.BlockSpec((RB, D), lambda i: (i % GRID, 0)),
            pl.BlockSpec((1, D, HH), lambda i: (i // GRID, 0, 0)),
            pl.BlockSpec((NTILES, RB), lambda i: (0, i % GRID)),
        ],
        out_specs=pl.BlockSpec((RB, HH), lambda i: (i, 0)),
        out_shape=jax.ShapeDtypeStruct((2 * NPAD, HH), jnp.float32),
    )(x_p, w1h, deg_parts)


def _tc2_body(agg_ref, deg_ref, b1_ref, w2_ref, o_ref):
    dinv = _dinv_of(deg_ref)
    a = jnp.concatenate([agg_ref[0], agg_ref[1]], axis=1)
    o1 = a * dinv[:, None] + b1_ref[...]
    r = jnp.maximum(o1, 0.0)
    h2 = jnp.dot(r, w2_ref[0], preferred_element_type=jnp.float32)
    o_ref[...] = h2 * dinv[:, None]


def _tc2(agg1, deg_parts, b1, w2h):
    return pl.pallas_call(
        _tc2_body,
        grid=(2 * GRID,),
        in_specs=[
            pl.BlockSpec((NC, RB, HH), lambda i: (0, i % GRID, 0)),
            pl.BlockSpec((NTILES, RB), lambda i: (0, i % GRID)),
            pl.BlockSpec((1, H), lambda i: (0, 0)),
            pl.BlockSpec((1, H, CH), lambda i: (i // GRID, 0, 0)),
        ],
        out_specs=pl.BlockSpec((RB, CH), lambda i: (i, 0)),
        out_shape=jax.ShapeDtypeStruct((2 * NPAD, CH), jnp.float32),
    )(agg1, deg_parts, b1, w2h)


def _tc3_body(agg_ref, deg_ref, b2_ref, o_ref):
    dinv = _dinv_of(deg_ref)
    a = jnp.concatenate([agg_ref[0], agg_ref[1]], axis=1)
    a = a * dinv[:, None] + b2_ref[...]
    col = lax.broadcasted_iota(jnp.int32, (RB, CP), 1)
    valid = col < C
    m = jnp.max(jnp.where(valid, a, -jnp.inf), axis=1, keepdims=True)
    e = jnp.where(valid, jnp.exp(a - m), 0.0)
    lse = jnp.log(jnp.sum(e, axis=1, keepdims=True))
    o_ref[...] = a - m - lse


def _tc3(agg2, deg_parts, b2p):
    return pl.pallas_call(
        _tc3_body,
        grid=(GRID,),
        in_specs=[
            pl.BlockSpec((NC, RB, CH), lambda i: (0, i, 0)),
            pl.BlockSpec((NTILES, RB), lambda i: (0, i)),
            pl.BlockSpec((1, CP), lambda i: (0, 0)),
        ],
        out_specs=pl.BlockSpec((RB, CP), lambda i: (i, 0)),
        out_shape=jax.ShapeDtypeStruct((NPAD, CP), jnp.float32),
    )(agg2, deg_parts, b2p)


def kernel(x, edge_index, W1, b1, W2, b2):
    src = edge_index[0]
    dst = edge_index[1]
    pad = jnp.full((EPAD - E,), N, dtype=jnp.int32)
    src_p = jnp.concatenate([src, pad])
    dst_p = jnp.concatenate([dst, pad])
    src_l1 = src_p.reshape(NS, EPT2 // CH1, CH1)
    src4_l1 = jnp.stack([src_l1, src_l1 + NPAD])
    dst3_l1 = dst_p.reshape(NS, EPT2 // CH1, CH1)
    src_l2 = src_p.reshape(NS, EPT2 // CH2, CH2)
    src4_l2 = jnp.stack([src_l2, src_l2 + NPAD])
    dst3_l2 = dst_p.reshape(NS, EPT2 // CH2, CH2)
    dst2 = dst_p.reshape(NTILES, EPT)

    x_p = jnp.zeros((NPAD, D), jnp.float32).at[:N].set(x)
    w1h = jnp.stack([W1[:, :HH], W1[:, HH:]])
    w2p = jnp.zeros((H, CP), jnp.float32).at[:, :C].set(W2)
    w2h = jnp.stack([w2p[:, :CH], w2p[:, CH:]])
    b2p = jnp.zeros((1, CP), jnp.float32).at[0, :C].set(b2)

    deg_parts = _deg_kernel(dst2)
    p1 = _tc1(x_p, w1h, deg_parts)
    agg1 = _scatter_l1(p1, src4_l1, dst3_l1)
    p2 = _tc2(agg1, deg_parts, b1.reshape(1, H), w2h)
    agg2 = _scatter_l2(p2, src4_l2, dst3_l2)
    out = _tc3(agg2, deg_parts, b2p)
    return out[:N, :C]


# speedup vs baseline: 1.0014x; 1.0014x over previous
import functools

import jax
import jax.numpy as jnp
from jax import lax
from jax.experimental import pallas as pl
from jax.experimental.pallas import tpu as pltpu
from jax.experimental.pallas import tpu_sc as plsc

N = 10000
E = 320000
D = 128
H = 128
C = 40

NC = 2
NS = 16
NTILES = NC * NS

NPAD = 10240
EPT = 10240
EPAD = NTILES * EPT
CHUNK = 128
EPT2 = EPAD // NS
NCH2 = EPT2 // CHUNK
ROWS_PER_TILE = NPAD // NS

HH = H // 2
CP = 64
CH = CP // 2
RB = 1024
GRID = NPAD // RB

_mesh = plsc.VectorSubcoreMesh(core_axis_name="c", subcore_axis_name="s")


@functools.partial(
    pl.kernel,
    out_type=jax.ShapeDtypeStruct((NTILES, NPAD), jnp.float32),
    mesh=_mesh,
    scratch_types=[
        pltpu.VMEM((EPT,), jnp.int32),
        pltpu.VMEM((NPAD,), jnp.float32),
    ],
    compiler_params=pltpu.CompilerParams(needs_layout_passes=False),
)
def _deg_kernel(dst_hbm, out_hbm, dst_v, hist_v):
    g = lax.axis_index("c") * NS + lax.axis_index("s")
    pltpu.sync_copy(dst_hbm.at[g], dst_v)
    zero16 = jnp.zeros((16,), jnp.float32)
    ones16 = jnp.ones((16,), jnp.float32)

    def zbody(i, _):
        hist_v[pl.ds(i * 16, 16)] = zero16
        return 0

    lax.fori_loop(0, NPAD // 16, zbody, 0)

    def hbody(i, _):
        idx = dst_v[pl.ds(i * 16, 16)]
        plsc.addupdate_scatter(hist_v, [idx], ones16)
        return 0

    lax.fori_loop(0, EPT // 16, hbody, 0)
    pltpu.sync_copy(hist_v, out_hbm.at[g])


def _make_scatter(width, G, chunk):
    nch = EPT2 // chunk
    ngroups = nch // G
    npairs = ngroups // 2
    assert EPT2 % chunk == 0 and nch % G == 0 and ngroups % 2 == 0
    assert npairs >= 2

    @functools.partial(
        pl.kernel,
        out_type=jax.ShapeDtypeStruct((NC, NPAD, width), jnp.float32),
        mesh=_mesh,
        scratch_types=[
            pltpu.VMEM((nch, chunk), jnp.int32),
            pltpu.VMEM((nch, chunk), jnp.int32),
            pltpu.VMEM((2 * G, chunk, width), jnp.float32),
            pltpu.VMEM_SHARED((NPAD, width), jnp.float32),
            pltpu.SemaphoreType.DMA,
            pltpu.SemaphoreType.DMA,
            pltpu.SemaphoreType.DMA,
            pltpu.SemaphoreType.DMA,
        ],
        compiler_params=pltpu.CompilerParams(use_tc_tiling_on_sc=False),
    )
    def _scatter(p_hbm, src_hbm, dst_hbm, out_hbm,
                 src_v, dst_v, rows_v, agg_sh, gs0, gs1, ss0, ss1):
        c = lax.axis_index("c")
        s = lax.axis_index("s")
        gsems = (gs0, gs1)
        ssems = (ss0, ss1)
        pltpu.sync_copy(src_hbm.at[c, s], src_v)
        pltpu.sync_copy(dst_hbm.at[s], dst_v)
        base = s * ROWS_PER_TILE
        pltpu.sync_copy(p_hbm.at[pl.ds(c * NPAD + base, ROWS_PER_TILE)],
                        agg_sh.at[pl.ds(base, ROWS_PER_TILE)])
        plsc.subcore_barrier()

        def fire_gathers(t, st):
            for b in range(G):
                pltpu.async_copy(p_hbm.at[src_v.at[t * G + b]],
                                 rows_v.at[st * G + b], gsems[st])

        def wait_gathers(t, st):
            for b in range(G):
                pltpu.make_async_copy(p_hbm.at[src_v.at[t * G + b]],
                                      rows_v.at[st * G + b], gsems[st]).wait()

        def fire_scatters(t, st):
            for b in range(G):
                pltpu.async_copy(rows_v.at[st * G + b],
                                 agg_sh.at[dst_v.at[t * G + b]], ssems[st],
                                 add=True)

        def wait_scatters(t, st):
            for b in range(G):
                pltpu.make_async_copy(rows_v.at[st * G + b],
                                      agg_sh.at[dst_v.at[t * G + b]],
                                      ssems[st]).wait()

        fire_gathers(0, 0)

        def obody(u, _):
            t0 = 2 * u
            t1 = 2 * u + 1
            wait_gathers(t0, 0)
            fire_scatters(t0, 0)

            @pl.when(u > 0)
            def _():
                wait_scatters(t1 - 2, 1)

            fire_gathers(t1, 1)
            wait_gathers(t1, 1)
            fire_scatters(t1, 1)
            wait_scatters(t0, 0)

            @pl.when(u < npairs - 1)
            def _():
                fire_gathers(t0 + 2, 0)

            return 0

        lax.fori_loop(0, npairs, obody, 0)
        wait_scatters(2 * npairs - 1, 1)
        plsc.subcore_barrier()
        pltpu.sync_copy(agg_sh.at[pl.ds(base, ROWS_PER_TILE)],
                        out_hbm.at[c, pl.ds(base, ROWS_PER_TILE)])

    return _scatter


CH1 = 80
CH2 = 128
_scatter_l1 = _make_scatter(HH, 4, CH1)
_scatter_l2 = _make_scatter(CH, 8, CH2)


def _dinv_of(deg_ref):
    deg = jnp.sum(deg_ref[...], axis=0) + 1.0
    return lax.rsqrt(deg)


def _tc1_body(x_ref, w_ref, deg_ref, o_ref):
    dinv = _dinv_of(deg_ref)
    h = jnp.dot(x_ref[...], w_ref[0], preferred_element_type=jnp.float32)
    o_ref[...] = h * dinv[:, None]


def _tc1(x_p, w1h, deg_parts):
    return pl.pallas_call(
        _tc1_body,
        grid=(2 * GRID,),
        in_specs=[
            pl.BlockSpec((RB, D), lambda i: (i % GRID, 0)),
            pl.BlockSpec((1, D, HH), lambda i: (i // GRID, 0, 0)),
            pl.BlockSpec((NTILES, RB), lambda i: (0, i % GRID)),
        ],
        out_specs=pl.BlockSpec((RB, HH), lambda i: (i, 0)),
        out_shape=jax.ShapeDtypeStruct((2 * NPAD, HH), jnp.float32),
    )(x_p, w1h, deg_parts)


def _tc2_body(agg_ref, deg_ref, b1_ref, w2_ref, o_ref):
    dinv = _dinv_of(deg_ref)
    a = jnp.concatenate([agg_ref[0], agg_ref[1]], axis=1)
    o1 = a * dinv[:, None] + b1_ref[...]
    r = jnp.maximum(o1, 0.0)
    h2 = jnp.dot(r, w2_ref[0], preferred_element_type=jnp.float32)
    o_ref[...] = h2 * dinv[:, None]


def _tc2(agg1, deg_parts, b1, w2h):
    return pl.pallas_call(
        _tc2_body,
        grid=(2 * GRID,),
        in_specs=[
            pl.BlockSpec((NC, RB, HH), lambda i: (0, i % GRID, 0)),
            pl.BlockSpec((NTILES, RB), lambda i: (0, i % GRID)),
            pl.BlockSpec((1, H), lambda i: (0, 0)),
            pl.BlockSpec((1, H, CH), lambda i: (i // GRID, 0, 0)),
        ],
        out_specs=pl.BlockSpec((RB, CH), lambda i: (i, 0)),
        out_shape=jax.ShapeDtypeStruct((2 * NPAD, CH), jnp.float32),
    )(agg1, deg_parts, b1, w2h)


def _tc3_body(agg_ref, deg_ref, b2_ref, o_ref):
    dinv = _dinv_of(deg_ref)
    a = jnp.concatenate([agg_ref[0], agg_ref[1]], axis=1)
    a = a * dinv[:, None] + b2_ref[...]
    col = lax.broadcasted_iota(jnp.int32, (RB, CP), 1)
    valid = col < C
    m = jnp.max(jnp.where(valid, a, -jnp.inf), axis=1, keepdims=True)
    e = jnp.where(valid, jnp.exp(a - m), 0.0)
    lse = jnp.log(jnp.sum(e, axis=1, keepdims=True))
    o_ref[...] = a - m - lse


def _tc3(agg2, deg_parts, b2p):
    return pl.pallas_call(
        _tc3_body,
        grid=(GRID,),
        in_specs=[
            pl.BlockSpec((NC, RB, CH), lambda i: (0, i, 0)),
            pl.BlockSpec((NTILES, RB), lambda i: (0, i)),
            pl.BlockSpec((1, CP), lambda i: (0, 0)),
        ],
        out_specs=pl.BlockSpec((RB, CP), lambda i: (i, 0)),
        out_shape=jax.ShapeDtypeStruct((NPAD, CP), jnp.float32),
    )(agg2, deg_parts, b2p)


def kernel(x, edge_index, W1, b1, W2, b2):
    src = edge_index[0]
    dst = edge_index[1]
    pad = jnp.full((EPAD - E,), N, dtype=jnp.int32)
    src_p = jnp.concatenate([src, pad])
    dst_p = jnp.concatenate([dst, pad])
    src_l1 = src_p.reshape(NS, EPT2 // CH1, CH1)
    src4_l1 = jnp.stack([src_l1, src_l1 + NPAD])
    dst3_l1 = dst_p.reshape(NS, EPT2 // CH1, CH1)
    src_l2 = src_p.reshape(NS, EPT2 // CH2, CH2)
    src4_l2 = jnp.stack([src_l2, src_l2 + NPAD])
    dst3_l2 = dst_p.reshape(NS, EPT2 // CH2, CH2)
    dst2 = dst_p.reshape(NTILES, EPT)

    x_p = jnp.zeros((NPAD, D), jnp.float32).at[:N].set(x)
    w1h = jnp.stack([W1[:, :HH], W1[:, HH:]])
    w2p = jnp.zeros((H, CP), jnp.float32).at[:, :C].set(W2)
    w2h = jnp.stack([w2p[:, :CH], w2p[:, CH:]])
    b2p = jnp.zeros((1, CP), jnp.float32).at[0, :C].set(b2)

    deg_parts = _deg_kernel(dst2)
    p1 = _tc1(x_p, w1h, deg_parts)
    agg1 = _scatter_l1(p1, src4_l1, dst3_l1)
    p2 = _tc2(agg1, deg_parts, b1.reshape(1, H), w2h)
    agg2 = _scatter_l2(p2, src4_l2, dst3_l2)
    out = _tc3(agg2, deg_parts, b2p)
    return out[:N, :C]
